# Initial kernel scaffold; baseline (speedup 1.0000x reference)
import jax
import jax.numpy as jnp
from jax.experimental import pallas as pl


def kernel(hm, hps, wh, reg, scale, gt_hm, gt_hps, gt_wh, gt_reg, gt_scale, hps_mask, reg_mask, ind):
    raise NotImplementedError("write your pallas kernel here")


# speedup vs baseline: 5.4112x; 5.4112x over previous
import functools

import jax
import jax.numpy as jnp
from jax import lax
from jax.experimental import pallas as pl
from jax.experimental.pallas import tpu as pltpu
from jax.experimental.pallas import tpu_sc as plsc

B, S, M, H, W = 16, 4, 10, 128, 128
HW = H * W
NW = 32
PAIRS_PER_W = 2

_V_HPS, _V_WH, _V_REG, _V_SC = 0, 384, 448, 512
_I_LEN = 576
_T_HPS, _M_HPS = 0, 384
_T_WH, _M_WH = 768, 832
_T_REG, _M_REG = 896, 960
_T_SC, _M_SC = 1024, 1088
_F_LEN = 1152


def _pack(x, pair_pad, region_len):
    x = x.reshape(B * S, -1)
    x = jnp.pad(x, ((0, 0), (0, pair_pad - x.shape[1])))
    x = x.reshape(NW, PAIRS_PER_W * pair_pad)
    if x.shape[1] < region_len:
        x = jnp.pad(x, ((0, 0), (0, region_len - x.shape[1])))
    return x


def _sc_reg_losses(aux_i, aux_f, hps_f, wh_f, reg_f, scale_f):
    mesh = plsc.VectorSubcoreMesh(core_axis_name="c", subcore_axis_name="s")

    @functools.partial(
        pl.kernel,
        mesh=mesh,
        out_type=jax.ShapeDtypeStruct((NW, 16), jnp.float32),
        scratch_types=[
            pltpu.VMEM((_I_LEN,), jnp.int32),
            pltpu.VMEM((_F_LEN,), jnp.float32),
            pltpu.VMEM((_I_LEN,), jnp.float32),
            pltpu.VMEM((16,), jnp.float32),
            pltpu.SemaphoreType.DMA,
        ],
        compiler_params=pltpu.CompilerParams(needs_layout_passes=False),
    )
    def sc_fn(auxi_hbm, auxf_hbm, hps_hbm, wh_hbm, reg_hbm, sc_hbm, out_hbm,
              idx_v, auxf_v, vals_v, outv, sem):
        wid = lax.axis_index("s") * 2 + lax.axis_index("c")
        pltpu.sync_copy(auxi_hbm.at[wid], idx_v)
        pltpu.sync_copy(auxf_hbm.at[wid], auxf_v)
        copies = []
        for j in range(3):
            copies.append(pltpu.async_copy(
                hps_hbm.at[idx_v.at[pl.ds(128 * j, 128)]],
                vals_v.at[pl.ds(128 * j, 128)], sem))
        for tbl, off in ((wh_hbm, _V_WH), (reg_hbm, _V_REG), (sc_hbm, _V_SC)):
            copies.append(pltpu.async_copy(
                tbl.at[idx_v.at[pl.ds(off, 64)]],
                vals_v.at[pl.ds(off, 64)], sem))
        for cp in copies:
            cp.wait()

        def l1(voff, toff, moff, ngroups):
            acc = jnp.zeros((16,), jnp.float32)
            mac = jnp.zeros((16,), jnp.float32)
            for g in range(ngroups):
                o = 16 * g
                t = auxf_v[pl.ds(toff + o, 16)]
                mm = auxf_v[pl.ds(moff + o, 16)]
                v = vals_v[pl.ds(voff + o, 16)]
                acc = acc + jnp.abs(t - v) * mm
                mac = mac + mm
            return jnp.sum(acc), jnp.sum(mac)

        res = []
        for p in range(PAIRS_PER_W):
            res += l1(_V_HPS + 160 * p, _T_HPS + 160 * p, _M_HPS + 160 * p, 10)
            res += l1(_V_WH + 32 * p, _T_WH + 32 * p, _M_WH + 32 * p, 2)
            res += l1(_V_REG + 32 * p, _T_REG + 32 * p, _M_REG + 32 * p, 2)
            res += l1(_V_SC + 32 * p, _T_SC + 32 * p, _M_SC + 32 * p, 2)
        lanes = lax.iota(jnp.int32, 16)
        vec = jnp.zeros((16,), jnp.float32)
        for i, v in enumerate(res):
            vec = jnp.where(lanes == i, v, vec)
        outv[...] = vec
        pltpu.sync_copy(outv, out_hbm.at[wid])

    return sc_fn(aux_i, aux_f, hps_f, wh_f, reg_f, scale_f)


def _tc_total(hm_r, gt_r, reg_losses):
    def body(reg_smem, hm_ref, gt_ref, out_ref):
        b = pl.program_id(0)
        x = hm_ref[0]
        pr = jnp.clip(jax.nn.sigmoid(x), 0.0001, 1.0 - 0.0001)
        om = 1.0 - pr
        lp = jnp.log(pr) * om * om
        ln = jnp.log(om) * pr * pr
        row = None
        for s in range(S):
            gt = gt_ref[0, s]
            pos = gt == 1.0
            neg = gt < 1.0
            w1 = 1.0 - gt
            w2 = w1 * w1
            w4 = w2 * w2
            pos_l = jnp.sum(jnp.where(pos, lp, 0.0))
            neg_l = jnp.sum(jnp.where(neg, ln * w4, 0.0))
            npos = jnp.sum(pos.astype(jnp.float32))
            zero = (npos == 0.0).astype(jnp.float32)
            hm_l = (-neg_l * zero
                    - (pos_l + neg_l) / (npos + zero) * (1.0 - zero))
            o = 8 * s
            hp_l = reg_smem[b, o + 0] / (reg_smem[b, o + 1] + 1e-4)
            wh_l = reg_smem[b, o + 2] / (reg_smem[b, o + 3] + 1e-4)
            off_l = reg_smem[b, o + 4] / (reg_smem[b, o + 5] + 1e-4)
            sc_l = reg_smem[b, o + 6] / (reg_smem[b, o + 7] + 1e-4)
            tot = hm_l + hp_l + 0.1 * wh_l + off_l + sc_l
            row = tot if row is None else jnp.minimum(row, tot)

        @pl.when(b == 0)
        def _():
            out_ref[0, 0] = 0.0

        out_ref[0, 0] += row * (1.0 / B)

    return pl.pallas_call(
        body,
        grid=(B,),
        in_specs=[
            pl.BlockSpec(memory_space=pltpu.SMEM),
            pl.BlockSpec((1, H, W), lambda b: (b, 0, 0)),
            pl.BlockSpec((1, S, H, W), lambda b: (b, 0, 0, 0)),
        ],
        out_specs=pl.BlockSpec(memory_space=pltpu.SMEM),
        out_shape=jax.ShapeDtypeStruct((1, 1), jnp.float32),
    )(reg_losses, hm_r, gt_r)


def kernel(hm, hps, wh, reg, scale, gt_hm, gt_hps, gt_wh, gt_reg, gt_scale,
           hps_mask, reg_mask, ind):
    ind = ind.astype(jnp.int32)

    def flat_idx(nc):
        boff = (jnp.arange(B, dtype=jnp.int32) * (nc * HW)).reshape(B, 1, 1, 1)
        coff = (jnp.arange(nc, dtype=jnp.int32) * HW).reshape(1, 1, 1, nc)
        return (ind[:, :, :, None] + coff + boff).reshape(B, S, M * nc)

    idx2 = flat_idx(2)
    aux_i = jnp.concatenate([
        _pack(flat_idx(16), 160, 384),
        _pack(idx2, 32, 64),
        _pack(idx2, 32, 64),
        _pack(flat_idx(3), 32, 64),
    ], axis=1)
    rm = reg_mask.astype(jnp.float32)
    rm2 = jnp.broadcast_to(rm[..., None], (B, S, M, 2))
    aux_f = jnp.concatenate([
        _pack(gt_hps, 160, 384),
        _pack(hps_mask.astype(jnp.float32), 160, 384),
        _pack(gt_wh, 32, 64),
        _pack(rm2, 32, 64),
        _pack(gt_reg, 32, 64),
        _pack(rm2, 32, 64),
        _pack(gt_scale, 32, 64),
        _pack(jnp.broadcast_to(rm[..., None], (B, S, M, 3)), 32, 64),
    ], axis=1)

    sc_out = _sc_reg_losses(aux_i, aux_f, hps.reshape(-1), wh.reshape(-1),
                            reg.reshape(-1), scale.reshape(-1))
    reg_sums = sc_out.reshape(B, S * 8)
    total = _tc_total(hm.reshape(B, H, W), gt_hm.reshape(B, S, H, W),
                      reg_sums)
    return total[0, 0]


# speedup vs baseline: 6.8259x; 1.2614x over previous
import functools

import jax
import jax.numpy as jnp
from jax import lax
from jax.experimental import pallas as pl
from jax.experimental.pallas import tpu as pltpu
from jax.experimental.pallas import tpu_sc as plsc

B, S, M, H, W = 16, 4, 10, 128, 128
HW = H * W
NW = 32
PAIRS_PER_W = 2

_V_HPS, _V_WH, _V_REG, _V_SC = 0, 384, 448, 512
_I_LEN = 576
_T_HPS, _M_HPS = 0, 384
_T_WH, _M_WH = 768, 832
_T_REG, _M_REG = 896, 960
_T_SC, _M_SC = 1024, 1088
_F_LEN = 1152


def _pack(x, pair_pad, region_len):
    x = x.reshape(B * S, -1)
    x = jnp.pad(x, ((0, 0), (0, pair_pad - x.shape[1])))
    x = x.reshape(NW, PAIRS_PER_W * pair_pad)
    if x.shape[1] < region_len:
        x = jnp.pad(x, ((0, 0), (0, region_len - x.shape[1])))
    return x


def _sc_reg_losses(aux_i, aux_f, hps_f, wh_f, reg_f, scale_f):
    mesh = plsc.VectorSubcoreMesh(core_axis_name="c", subcore_axis_name="s")

    @functools.partial(
        pl.kernel,
        mesh=mesh,
        out_type=jax.ShapeDtypeStruct((NW, 16), jnp.float32),
        scratch_types=[
            pltpu.VMEM((_I_LEN,), jnp.int32),
            pltpu.VMEM((_F_LEN,), jnp.float32),
            pltpu.VMEM((_I_LEN,), jnp.float32),
            pltpu.VMEM((16,), jnp.float32),
            pltpu.SemaphoreType.DMA,
            pltpu.SemaphoreType.DMA,
        ],
        compiler_params=pltpu.CompilerParams(needs_layout_passes=False),
    )
    def sc_fn(auxi_hbm, auxf_hbm, hps_hbm, wh_hbm, reg_hbm, sc_hbm, out_hbm,
              idx_v, auxf_v, vals_v, outv, sem, semf):
        wid = lax.axis_index("s") * 2 + lax.axis_index("c")
        cpf = pltpu.async_copy(auxf_hbm.at[wid], auxf_v, semf)
        pltpu.sync_copy(auxi_hbm.at[wid], idx_v)
        copies = []
        for j in range(3):
            copies.append(pltpu.async_copy(
                hps_hbm.at[idx_v.at[pl.ds(128 * j, 128)]],
                vals_v.at[pl.ds(128 * j, 128)], sem))
        for tbl, off in ((wh_hbm, _V_WH), (reg_hbm, _V_REG), (sc_hbm, _V_SC)):
            copies.append(pltpu.async_copy(
                tbl.at[idx_v.at[pl.ds(off, 64)]],
                vals_v.at[pl.ds(off, 64)], sem))
        for cp in copies:
            cp.wait()
        cpf.wait()

        def l1(voff, toff, moff, ngroups):
            acc = jnp.zeros((16,), jnp.float32)
            mac = jnp.zeros((16,), jnp.float32)
            for g in range(ngroups):
                o = 16 * g
                t = auxf_v[pl.ds(toff + o, 16)]
                mm = auxf_v[pl.ds(moff + o, 16)]
                v = vals_v[pl.ds(voff + o, 16)]
                acc = acc + jnp.abs(t - v) * mm
                mac = mac + mm
            return jnp.sum(acc), jnp.sum(mac)

        res = []
        for p in range(PAIRS_PER_W):
            res += l1(_V_HPS + 160 * p, _T_HPS + 160 * p, _M_HPS + 160 * p, 10)
            res += l1(_V_WH + 32 * p, _T_WH + 32 * p, _M_WH + 32 * p, 2)
            res += l1(_V_REG + 32 * p, _T_REG + 32 * p, _M_REG + 32 * p, 2)
            res += l1(_V_SC + 32 * p, _T_SC + 32 * p, _M_SC + 32 * p, 2)
        lanes = lax.iota(jnp.int32, 16)
        vec = jnp.zeros((16,), jnp.float32)
        for i, v in enumerate(res):
            vec = jnp.where(lanes == i, v, vec)
        outv[...] = vec
        pltpu.sync_copy(outv, out_hbm.at[wid])

    return sc_fn(aux_i, aux_f, hps_f, wh_f, reg_f, scale_f)


def _tc_focal(hm_r, gt_r):
    def body(hm_ref, gt_ref, out_ref):
        b = pl.program_id(0)
        x = hm_ref[0]
        pr = jnp.clip(jax.nn.sigmoid(x), 0.0001, 1.0 - 0.0001)
        om = 1.0 - pr
        lp = jnp.log(pr) * om * om
        ln = jnp.log(om) * pr * pr
        for s in range(S):
            gt = gt_ref[0, s]
            pos = gt == 1.0
            neg = gt < 1.0
            w1 = 1.0 - gt
            w2 = w1 * w1
            w4 = w2 * w2
            pos_l = jnp.sum(jnp.where(pos, lp, 0.0))
            neg_l = jnp.sum(jnp.where(neg, ln * w4, 0.0))
            npos = jnp.sum(pos.astype(jnp.float32))
            zero = (npos == 0.0).astype(jnp.float32)
            out_ref[b, s] = (-neg_l * zero
                             - (pos_l + neg_l) / (npos + zero) * (1.0 - zero))

    return pl.pallas_call(
        body,
        grid=(B,),
        in_specs=[
            pl.BlockSpec((1, H, W), lambda b: (b, 0, 0)),
            pl.BlockSpec((1, S, H, W), lambda b: (b, 0, 0, 0)),
        ],
        out_specs=pl.BlockSpec(memory_space=pltpu.SMEM),
        out_shape=jax.ShapeDtypeStruct((B, S), jnp.float32),
    )(hm_r, gt_r)


def _tc_combine(hm_ls, sc_out):
    def body(hm_smem, reg_smem, out_ref):
        acc = 0.0
        for b in range(B):
            row = None
            for s in range(S):
                w = (b * S + s) // 2
                o = 8 * ((b * S + s) % 2)
                hp_l = reg_smem[w, o + 0] / (reg_smem[w, o + 1] + 1e-4)
                wh_l = reg_smem[w, o + 2] / (reg_smem[w, o + 3] + 1e-4)
                off_l = reg_smem[w, o + 4] / (reg_smem[w, o + 5] + 1e-4)
                sc_l = reg_smem[w, o + 6] / (reg_smem[w, o + 7] + 1e-4)
                tot = hm_smem[b, s] + hp_l + 0.1 * wh_l + off_l + sc_l
                row = tot if row is None else jnp.minimum(row, tot)
            acc = acc + row * (1.0 / B)
        out_ref[0, 0] = acc

    return pl.pallas_call(
        body,
        in_specs=[
            pl.BlockSpec(memory_space=pltpu.SMEM),
            pl.BlockSpec(memory_space=pltpu.SMEM),
        ],
        out_specs=pl.BlockSpec(memory_space=pltpu.SMEM),
        out_shape=jax.ShapeDtypeStruct((1, 1), jnp.float32),
    )(hm_ls, sc_out)


def kernel(hm, hps, wh, reg, scale, gt_hm, gt_hps, gt_wh, gt_reg, gt_scale,
           hps_mask, reg_mask, ind):
    ind = ind.astype(jnp.int32)

    def flat_idx(nc):
        boff = (jnp.arange(B, dtype=jnp.int32) * (nc * HW)).reshape(B, 1, 1, 1)
        coff = (jnp.arange(nc, dtype=jnp.int32) * HW).reshape(1, 1, 1, nc)
        return (ind[:, :, :, None] + coff + boff).reshape(B, S, M * nc)

    idx2 = flat_idx(2)
    aux_i = jnp.concatenate([
        _pack(flat_idx(16), 160, 384),
        _pack(idx2, 32, 64),
        _pack(idx2, 32, 64),
        _pack(flat_idx(3), 32, 64),
    ], axis=1)
    rm = reg_mask.astype(jnp.float32)
    rm2 = jnp.broadcast_to(rm[..., None], (B, S, M, 2))
    aux_f = jnp.concatenate([
        _pack(gt_hps, 160, 384),
        _pack(hps_mask.astype(jnp.float32), 160, 384),
        _pack(gt_wh, 32, 64),
        _pack(rm2, 32, 64),
        _pack(gt_reg, 32, 64),
        _pack(rm2, 32, 64),
        _pack(gt_scale, 32, 64),
        _pack(jnp.broadcast_to(rm[..., None], (B, S, M, 3)), 32, 64),
    ], axis=1)

    sc_out = _sc_reg_losses(aux_i, aux_f, hps.reshape(-1), wh.reshape(-1),
                            reg.reshape(-1), scale.reshape(-1))
    hm_ls = _tc_focal(hm.reshape(B, H, W), gt_hm.reshape(B, S, H, W))
    total = _tc_combine(hm_ls, sc_out)
    return total[0, 0]
